# baseline (speedup 1.0000x reference)
import dataclasses
import functools

import jax
import jax.numpy as jnp
from jax import lax
from jax.experimental import pallas as pl
from jax.experimental.pallas import tpu as pltpu
from jax.experimental.pallas import tpu_sc as plsc

N = 10000
E = 320000
D = 128
HEADS = 8
OC = 16
ETE = 10
EAE = 50
STEPS = 6
NEG = 0.2

NC = 2
NS = 16
NW = NC * NS
NPT = 320
NPL = 20
NPAD = NW * NPT
W = 32
L = 16
WB = W * L

NEGBIG = -3.0e38


def _full(v):
    return jnp.full((L,), v, dtype=jnp.int32)


def _edge_kernel_body(nf32_hbm, dst_hbm, src_hbm, et_hbm, ea_hbm,
                      lest_hbm, lcnt_hbm, tab_hbm, out_hbm,
                      lest_v, lcnt_v, tab_v, ai_v, amax_v, den_v, acc_v,
                      ibuf, dst_b, src_b, et_b, ea_b, g2_c):
    wid = lax.axis_index("s") * NC + lax.axis_index("c")
    pltpu.sync_copy(lest_hbm, lest_v)
    pltpu.sync_copy(lcnt_hbm, lcnt_v)
    pltpu.sync_copy(tab_hbm, tab_v)
    lanes = lax.iota(jnp.int32, L)
    lo_v = lest_v[pl.ds(wid * L, L)]
    cnt = lcnt_v[pl.ds(wid * L, L)]
    maxcnt = jnp.max(cnt)
    nwin = (maxcnt + W - 1) // W
    cvec = tab_v[pl.ds(24, L)]
    dpvec = tab_v[pl.ds(40, L)]
    dnvec = tab_v[pl.ds(56, L)]
    n_lo = wid * NPT
    pltpu.sync_copy(nf32_hbm.at[pl.ds(n_lo, NPT)], ai_v)

    zf = jnp.zeros((L,), jnp.float32)
    negbig = jnp.full((L,), NEGBIG, jnp.float32)

    @pl.loop(0, (NPT * 8 + L) // L)
    def _(i):
        amax_v[pl.ds(i * L, L)] = negbig
        den_v[pl.ds(i * L, L)] = zf

    @pl.loop(0, NPT * 128 // L)
    def _(i):
        acc_v[pl.ds(i * L, L)] = zf

    def stage_window(w):
        @pl.loop(0, W)
        def _(i):
            p = jnp.clip(lo_v + (w * W + i), 0, E - 1)
            ibuf[pl.ds(i * L, L)] = p
        pltpu.sync_copy(dst_hbm.at[ibuf], dst_b)
        pltpu.sync_copy(src_hbm.at[ibuf], src_b)
        pltpu.sync_copy(et_hbm.at[ibuf], et_b)
        pltpu.sync_copy(ea_hbm.at[ibuf], ea_b)
        pltpu.sync_copy(nf32_hbm.at[src_b], g2_c)

    def edge_fields(w, i):
        k = w * W + i
        active = k < cnt
        r = i * L + lanes
        d = dst_b[pl.ds(i * L, L)]
        et_v = et_b[pl.ds(i * L, L)]
        ea_v = ea_b[pl.ds(i * L, L)]
        dlc = jnp.clip(d - n_lo, 0, NPT - 1)
        sa = ea_v >= 0.0
        return active, r, d, et_v, ea_v, dlc, sa

    def pre_h(h, dlc, r, et8, ea_v, sa):
        a_i = plsc.load_gather(ai_v, [dlc, _full(h)])
        a_j = plsc.load_gather(g2_c, [r, _full(8 + h)])
        th = plsc.load_gather(tab_v, [et8 + h])
        cs = jnp.where(sa, cvec[h], cvec[8 + h])
        p = a_i + a_j + th + ea_v * cs
        return jnp.where(p >= 0.0, p, NEG * p)

    def p1_win(w, carry):
        stage_window(w)

        def p1_it(i, carry):
            prev_d = carry[0]
            ms = carry[1:]
            active, r, d, et_v, ea_v, dlc, sa = edge_fields(w, i)
            d_eff = jnp.where(active, d, prev_d)
            chg = d_eff != prev_d
            fl = chg & (prev_d >= 0)
            pdlc = jnp.clip(prev_d - n_lo, 0, NPT - 1)
            et8 = et_v * 8
            new_ms = []
            for h in range(8):
                plsc.store_scatter(amax_v, [pdlc * 8 + h], ms[h], mask=fl)
                p = pre_h(h, dlc, r, et8, ea_v, sa)
                p = jnp.where(active, p, NEGBIG)
                base = jnp.where(chg, negbig, ms[h])
                new_ms.append(jnp.maximum(base, p))
            return (d_eff,) + tuple(new_ms)

        return lax.fori_loop(0, W, p1_it, carry)

    carry0 = (jnp.full((L,), -1, jnp.int32),) + tuple(negbig for _ in range(8))
    carry = lax.fori_loop(0, nwin, p1_win, carry0)
    prev_d = carry[0]
    fl = prev_d >= 0
    pdlc = jnp.clip(prev_d - n_lo, 0, NPT - 1)
    for h in range(8):
        plsc.store_scatter(amax_v, [pdlc * 8 + h], carry[1 + h], mask=fl)

    def p2_win(w, carry):
        stage_window(w)

        def p2_it(i, carry):
            active, r, d, et_v, ea_v, dlc, sa = edge_fields(w, i)
            et8 = et_v * 8
            exs = []
            for h in range(8):
                p = pre_h(h, dlc, r, et8, ea_v, sa)
                ch = plsc.load_gather(amax_v, [dlc * 8 + h])
                ex = jnp.exp(p - ch)
                ex = jnp.where(active, ex, 0.0)
                plsc.addupdate_scatter(den_v, [dlc * 8 + h], ex)
                exs.append(ex)
            bidx = dlc * 128
            for o in range(16):
                mcol = plsc.load_gather(g2_c, [r, _full(16 + o)])
                ds_ = jnp.where(sa, dpvec[o], dnvec[o])
                msg = mcol + ea_v * ds_
                for h in range(8):
                    plsc.addupdate_scatter(acc_v, [bidx + (h * 16 + o)],
                                           exs[h] * msg)
            return carry

        return lax.fori_loop(0, W, p2_it, carry)

    lax.fori_loop(0, nwin, p2_win, 0)

    @pl.loop(0, NPT)
    def _(dl):
        den16 = den_v[pl.ds(dl * 8, L)]
        r = 1.0 / (den16 + 1e-16)
        rs0 = jnp.broadcast_to(r[0], (L,))
        for h in range(8):
            rs = jnp.broadcast_to(r[h], (L,)) if h else rs0
            i0 = dl * 128 + h * 16
            acc_v[pl.ds(i0, L)] = acc_v[pl.ds(i0, L)] * rs

    pltpu.sync_copy(acc_v, out_hbm.at[pl.ds(n_lo * 128, NPT * 128)])


@jax.jit
def _edge_phase(nf32, dst_s, src_s, et_s, ea_s, lest, lcnt, tab):
    mesh = plsc.VectorSubcoreMesh(core_axis_name="c", subcore_axis_name="s")
    cp = pltpu.CompilerParams()
    if "needs_layout_passes" in pltpu.CompilerParams.__dataclass_fields__:
        cp = dataclasses.replace(cp, needs_layout_passes=False)
    if "use_tc_tiling_on_sc" in pltpu.CompilerParams.__dataclass_fields__:
        cp = dataclasses.replace(cp, use_tc_tiling_on_sc=False)
    if "disable_bounds_checks" in pltpu.CompilerParams.__dataclass_fields__:
        cp = dataclasses.replace(cp, disable_bounds_checks=True)
    f = pl.kernel(
        _edge_kernel_body,
        out_type=jax.ShapeDtypeStruct((NPAD * 128,), jnp.float32),
        mesh=mesh,
        compiler_params=cp,
        scratch_types=[
            pltpu.VMEM((NW * L + 16,), jnp.int32),
            pltpu.VMEM((NW * L + 16,), jnp.int32),
            pltpu.VMEM((80,), jnp.float32),
            pltpu.VMEM((NPT, 32), jnp.float32),
            pltpu.VMEM((NPT * 8 + L,), jnp.float32),
            pltpu.VMEM((NPT * 8 + L,), jnp.float32),
            pltpu.VMEM((NPT * 128,), jnp.float32),
            pltpu.VMEM((WB,), jnp.int32),
            pltpu.VMEM((WB,), jnp.int32),
            pltpu.VMEM((WB,), jnp.int32),
            pltpu.VMEM((WB,), jnp.int32),
            pltpu.VMEM((WB,), jnp.float32),
            pltpu.VMEM((WB, 32), jnp.float32),
        ],
    )
    return f(nf32, dst_s, src_s, et_s, ea_s, lest, lcnt, tab)


def _layer_norm(h, g, b):
    mu = jnp.mean(h, axis=-1, keepdims=True)
    var = jnp.mean((h - mu) ** 2, axis=-1, keepdims=True)
    return (h - mu) / jnp.sqrt(var + 1e-5) * g + b


def kernel(x, edge_index, edge_attr, node_type, edge_type, params):
    p = params
    src = edge_index[0].astype(jnp.int32)
    dst = edge_index[1].astype(jnp.int32)
    et = edge_type.astype(jnp.int32)
    nt = node_type.astype(jnp.int32)
    ea = edge_attr.astype(jnp.float32)

    order = jnp.argsort(dst)
    dst_s = dst[order]
    src_s = src[order]
    et_s = et[order]
    ea_s = ea[order]
    lest0 = jnp.searchsorted(
        dst_s, jnp.arange(NW * L + 1, dtype=jnp.int32) * NPL).astype(jnp.int32)
    lcnt = jnp.pad(lest0[1:] - lest0[:-1], (0, 16), constant_values=0)
    lest = jnp.pad(lest0[:-1], (0, 16), constant_values=E)

    Wi = p['att_W'][:OC]
    Wj = p['att_W'][OC:2 * OC]
    We = p['att_W'][2 * OC:2 * OC + ETE]
    Wa = p['att_W'][2 * OC + ETE:]
    Lx = p['lin_W'][:OC]
    Le = p['lin_W'][OC:]
    aW = p['edge_attr_W'][0]

    T = jnp.where(p['edge_type_emb'] >= 0, p['edge_type_emb'],
                  NEG * p['edge_type_emb']) @ We
    cpos = jnp.where(aW >= 0, aW, NEG * aW) @ Wa
    cneg = jnp.where(aW <= 0, aW, NEG * aW) @ Wa
    dpos = jnp.where(aW >= 0, aW, NEG * aW) @ Le
    dneg = jnp.where(aW <= 0, aW, NEG * aW) @ Le
    tab = jnp.concatenate([T.reshape(-1), cpos, cneg, dpos, dneg,
                           jnp.zeros((8,), jnp.float32)])

    M = jnp.concatenate([Wi, Wj, Lx], axis=1)
    G = jnp.einsum('tdo,oc->tdc', p['hetero_W'], M)
    gb = p['hetero_b'] @ M

    nt_pad = jnp.pad(nt, (0, NPAD - N))
    G_n = G[nt_pad]
    gb_n = gb[nt_pad]

    m = jnp.pad(x, ((0, NPAD - N), (0, 0)))
    for i in range(STEPS):
        nf32 = jnp.einsum('nd,ndc->nc', m, G_n) + gb_n
        h1 = _edge_phase(nf32, dst_s, src_s, et_s, ea_s, lest, lcnt, tab)
        h = h1.reshape(NPAD, 128)
        m = _layer_norm(h + m, p['ln1_g'][i], p['ln1_b'][i])
        f = jnp.maximum(m @ p['ffn_W1'][i] + p['ffn_b1'][i], 0.0) \
            @ p['ffn_W2'][i] + p['ffn_b2'][i]
        m = _layer_norm(f + m, p['ln2_g'][i], p['ln2_b'][i])
    return m[:N]


# speedup vs baseline: 1.6468x; 1.6468x over previous
import dataclasses
import functools

import jax
import jax.numpy as jnp
from jax import lax
from jax.experimental import pallas as pl
from jax.experimental.pallas import tpu as pltpu
from jax.experimental.pallas import tpu_sc as plsc

N = 10000
E = 320000
D = 128
HEADS = 8
OC = 16
ETE = 10
EAE = 50
STEPS = 6
NEG = 0.2

NC = 2
NS = 16
NW = NC * NS
NPT = 320
NPL = 20
NPAD = NW * NPT
W = 32
L = 16
WB = W * L

SLOT = 21
NPS = L * SLOT

NEGBIG = -3.0e38


def _full(v):
    return jnp.full((L,), v, dtype=jnp.int32)


def _edge_kernel_body(nf32_hbm, nf8t_hbm, dst_hbm, src_hbm, et_hbm, ea_hbm,
                      lest_hbm, lcnt_hbm, tab_hbm, out_hbm,
                      lest_v, lcnt_v, tab_v, ai8_v, amax_v, den_v, acc_v,
                      ibuf, dst_b, src_b, et_b, ea_b, g2_c):
    wid = lax.axis_index("s") * NC + lax.axis_index("c")
    pltpu.sync_copy(lest_hbm, lest_v)
    pltpu.sync_copy(lcnt_hbm, lcnt_v)
    pltpu.sync_copy(tab_hbm, tab_v)
    lanes = lax.iota(jnp.int32, L)
    lo_v = lest_v[pl.ds(wid * L, L)]
    cnt = lcnt_v[pl.ds(wid * L, L)]
    maxcnt = jnp.max(cnt)
    nwin = (maxcnt + W - 1) // W
    cvec = tab_v[pl.ds(24, L)]
    dpvec = tab_v[pl.ds(40, L)]
    dnvec = tab_v[pl.ds(56, L)]
    n_lo = wid * NPT
    pltpu.sync_copy(nf8t_hbm.at[wid], ai8_v)

    zf = jnp.zeros((L,), jnp.float32)
    negbig = jnp.full((L,), NEGBIG, jnp.float32)

    @pl.loop(0, NPS * 8 // L)
    def _(i):
        amax_v[pl.ds(i * L, L)] = negbig
        den_v[pl.ds(i * L, L)] = zf

    @pl.loop(0, NPS * 128 // L)
    def _(i):
        acc_v[pl.ds(i * L, L)] = zf

    def stage_window(w):
        @pl.loop(0, W)
        def _(i):
            p = jnp.clip(lo_v + (w * W + i), 0, E - 1)
            ibuf[pl.ds(i * L, L)] = p
        pltpu.sync_copy(dst_hbm.at[ibuf], dst_b)
        pltpu.sync_copy(src_hbm.at[ibuf], src_b)
        pltpu.sync_copy(et_hbm.at[ibuf], et_b)
        pltpu.sync_copy(ea_hbm.at[ibuf], ea_b)
        pltpu.sync_copy(nf32_hbm.at[src_b], g2_c)

    def edge_fields(w, i):
        k = w * W + i
        active = k < cnt
        r = i * L + lanes
        d = dst_b[pl.ds(i * L, L)]
        et_v = et_b[pl.ds(i * L, L)]
        ea_v = ea_b[pl.ds(i * L, L)]
        dlc = jnp.clip(d - n_lo, 0, NPT - 1) + lanes
        sa = ea_v >= 0.0
        return active, r, d, et_v, ea_v, dlc, sa

    def pre_h(h, dlc, r, et8, ea_v, sa):
        a_i = plsc.load_gather(ai8_v, [h * NPS + dlc])
        a_j = plsc.load_gather(g2_c, [r, _full(8 + h)])
        th = plsc.load_gather(tab_v, [et8 + h])
        cs = jnp.where(sa, cvec[h], cvec[8 + h])
        p = a_i + a_j + th + ea_v * cs
        return jnp.where(p >= 0.0, p, NEG * p)

    def p1_win(w, carry):
        stage_window(w)

        def p1_it(i, carry):
            prev_d = carry[0]
            ms = carry[1:]
            active, r, d, et_v, ea_v, dlc, sa = edge_fields(w, i)
            d_eff = jnp.where(active, d, prev_d)
            chg = d_eff != prev_d
            fl = chg & (prev_d >= 0)
            pdlc = jnp.clip(prev_d - n_lo, 0, NPT - 1) + lanes
            et8 = et_v * 8
            new_ms = []
            for h in range(8):
                plsc.store_scatter(amax_v, [h * NPS + pdlc], ms[h], mask=fl)
                p = pre_h(h, dlc, r, et8, ea_v, sa)
                p = jnp.where(active, p, NEGBIG)
                base = jnp.where(chg, negbig, ms[h])
                new_ms.append(jnp.maximum(base, p))
            return (d_eff,) + tuple(new_ms)

        return lax.fori_loop(0, W, p1_it, carry)

    carry0 = (jnp.full((L,), -1, jnp.int32),) + tuple(negbig for _ in range(8))
    carry = lax.fori_loop(0, nwin, p1_win, carry0)
    prev_d = carry[0]
    fl = prev_d >= 0
    pdlc = jnp.clip(prev_d - n_lo, 0, NPT - 1) + lanes
    for h in range(8):
        plsc.store_scatter(amax_v, [h * NPS + pdlc], carry[1 + h], mask=fl)

    def p2_win(w, carry):
        stage_window(w)

        def p2_it(i, carry):
            active, r, d, et_v, ea_v, dlc, sa = edge_fields(w, i)
            et8 = et_v * 8
            exs = []
            for h in range(8):
                p = pre_h(h, dlc, r, et8, ea_v, sa)
                ch = plsc.load_gather(amax_v, [h * NPS + dlc])
                ex = jnp.exp(p - ch)
                ex = jnp.where(active, ex, 0.0)
                plsc.addupdate_scatter(den_v, [h * NPS + dlc], ex)
                exs.append(ex)
            bidx = dlc
            for o in range(16):
                mcol = plsc.load_gather(g2_c, [r, _full(16 + o)])
                ds_ = jnp.where(sa, dpvec[o], dnvec[o])
                msg = mcol + ea_v * ds_
                for h in range(8):
                    plsc.addupdate_scatter(acc_v, [bidx + (h * 16 + o) * NPS],
                                           exs[h] * msg)
            return carry

        return lax.fori_loop(0, W, p2_it, carry)

    lax.fori_loop(0, nwin, p2_win, 0)

    @pl.loop(0, NPS * 8 // L)
    def _(i):
        den_v[pl.ds(i * L, L)] = 1.0 / (den_v[pl.ds(i * L, L)] + 1e-16)

    @pl.loop(0, 128)
    def _(ho):
        h = ho // 16
        for k in range(SLOT):
            ia = ho * NPS + k * L
            ib = h * NPS + k * L
            acc_v[pl.ds(ia, L)] = acc_v[pl.ds(ia, L)] * den_v[pl.ds(ib, L)]

    pltpu.sync_copy(acc_v, out_hbm.at[pl.ds(wid * (128 * NPS), 128 * NPS)])


@jax.jit
def _edge_phase(nf32, nf8t, dst_s, src_s, et_s, ea_s, lest, lcnt, tab):
    mesh = plsc.VectorSubcoreMesh(core_axis_name="c", subcore_axis_name="s")
    cp = pltpu.CompilerParams()
    if "needs_layout_passes" in pltpu.CompilerParams.__dataclass_fields__:
        cp = dataclasses.replace(cp, needs_layout_passes=False)
    if "use_tc_tiling_on_sc" in pltpu.CompilerParams.__dataclass_fields__:
        cp = dataclasses.replace(cp, use_tc_tiling_on_sc=False)
    if "disable_bounds_checks" in pltpu.CompilerParams.__dataclass_fields__:
        cp = dataclasses.replace(cp, disable_bounds_checks=True)
    f = pl.kernel(
        _edge_kernel_body,
        out_type=jax.ShapeDtypeStruct((NW * 128 * NPS,), jnp.float32),
        mesh=mesh,
        compiler_params=cp,
        scratch_types=[
            pltpu.VMEM((NW * L + 16,), jnp.int32),
            pltpu.VMEM((NW * L + 16,), jnp.int32),
            pltpu.VMEM((80,), jnp.float32),
            pltpu.VMEM((8 * NPS,), jnp.float32),
            pltpu.VMEM((8 * NPS + L,), jnp.float32),
            pltpu.VMEM((8 * NPS + L,), jnp.float32),
            pltpu.VMEM((128 * NPS,), jnp.float32),
            pltpu.VMEM((WB,), jnp.int32),
            pltpu.VMEM((WB,), jnp.int32),
            pltpu.VMEM((WB,), jnp.int32),
            pltpu.VMEM((WB,), jnp.int32),
            pltpu.VMEM((WB,), jnp.float32),
            pltpu.VMEM((WB, 32), jnp.float32),
        ],
    )
    return f(nf32, nf8t, dst_s, src_s, et_s, ea_s, lest, lcnt, tab)


def _layer_norm(h, g, b):
    mu = jnp.mean(h, axis=-1, keepdims=True)
    var = jnp.mean((h - mu) ** 2, axis=-1, keepdims=True)
    return (h - mu) / jnp.sqrt(var + 1e-5) * g + b


def kernel(x, edge_index, edge_attr, node_type, edge_type, params):
    p = params
    src = edge_index[0].astype(jnp.int32)
    dst = edge_index[1].astype(jnp.int32)
    et = edge_type.astype(jnp.int32)
    nt = node_type.astype(jnp.int32)
    ea = edge_attr.astype(jnp.float32)

    order = jnp.argsort(dst)
    dst_s = dst[order]
    src_s = src[order]
    et_s = et[order]
    ea_s = ea[order]
    lest0 = jnp.searchsorted(
        dst_s, jnp.arange(NW * L + 1, dtype=jnp.int32) * NPL).astype(jnp.int32)
    lcnt = jnp.pad(lest0[1:] - lest0[:-1], (0, 16), constant_values=0)
    lest = jnp.pad(lest0[:-1], (0, 16), constant_values=E)

    Wi = p['att_W'][:OC]
    Wj = p['att_W'][OC:2 * OC]
    We = p['att_W'][2 * OC:2 * OC + ETE]
    Wa = p['att_W'][2 * OC + ETE:]
    Lx = p['lin_W'][:OC]
    Le = p['lin_W'][OC:]
    aW = p['edge_attr_W'][0]

    T = jnp.where(p['edge_type_emb'] >= 0, p['edge_type_emb'],
                  NEG * p['edge_type_emb']) @ We
    cpos = jnp.where(aW >= 0, aW, NEG * aW) @ Wa
    cneg = jnp.where(aW <= 0, aW, NEG * aW) @ Wa
    dpos = jnp.where(aW >= 0, aW, NEG * aW) @ Le
    dneg = jnp.where(aW <= 0, aW, NEG * aW) @ Le
    tab = jnp.concatenate([T.reshape(-1), cpos, cneg, dpos, dneg,
                           jnp.zeros((8,), jnp.float32)])

    M = jnp.concatenate([Wi, Wj, Lx], axis=1)
    G = jnp.einsum('tdo,oc->tdc', p['hetero_W'], M)
    gb = p['hetero_b'] @ M

    nt_pad = jnp.pad(nt, (0, NPAD - N))
    G_n = G[nt_pad]
    gb_n = gb[nt_pad]

    m = jnp.pad(x, ((0, NPAD - N), (0, 0)))
    for i in range(STEPS):
        nf32 = jnp.einsum('nd,ndc->nc', m, G_n) + gb_n
        a8 = nf32[:, :8].reshape(NW, L, NPL, 8)
        a8 = jnp.pad(a8, ((0, 0), (0, 0), (0, 1), (0, 0)))
        nf8t = a8.transpose(0, 3, 1, 2).reshape(NW, 8 * NPS)
        h1 = _edge_phase(nf32, nf8t, dst_s, src_s, et_s, ea_s,
                         lest, lcnt, tab)
        h = h1.reshape(NW, 128, L, SLOT)[:, :, :, :NPL]
        h = h.transpose(0, 2, 3, 1).reshape(NPAD, 128)
        m = _layer_norm(h + m, p['ln1_g'][i], p['ln1_b'][i])
        f = jnp.maximum(m @ p['ffn_W1'][i] + p['ffn_b1'][i], 0.0) \
            @ p['ffn_W2'][i] + p['ffn_b2'][i]
        m = _layer_norm(f + m, p['ln2_g'][i], p['ln2_b'][i])
    return m[:N]
